# Initial kernel scaffold; baseline (speedup 1.0000x reference)
import jax
import jax.numpy as jnp
from jax.experimental import pallas as pl


def kernel(u_hyp, r_centres, angular_weight):
    raise NotImplementedError("write your pallas kernel here")


# speedup vs baseline: 4.8931x; 4.8931x over previous
import functools

import jax
import jax.numpy as jnp
from jax.experimental import pallas as pl

N_E = 8192
E_DIM = 64
BETA = 0.25
RADIAL_BINS = 16
ANGULAR_BINS = N_E // RADIAL_BINS
MAX_RADIUS = 1.1
TOP_R = 3
TOP_W = 5
BLK = 1024

_HIGH = jax.lax.Precision.HIGHEST


def _acosh(x):
    return jnp.log(x + jnp.sqrt((x - 1.0) * (x + 1.0)))


def _vq_kernel(x_ref, rc_ref, a64_ref, zq_ref, hist_ref, tad_ref, loss_ref,
               emean_ref, div_ref, perp_ref, *, nblk, n):
    i = pl.program_id(0)
    x = x_ref[...]
    u_t = x[:, 0:1]
    msq = jnp.maximum(jnp.sum(x * x, axis=1, keepdims=True) - u_t * u_t, 0.0)
    r = _acosh(jnp.maximum(u_t, 1.01))

    rc = jnp.clip(rc_ref[...], 0.01, MAX_RADIUS)

    dist_r = jnp.abs(r - rc)
    iota_r = jax.lax.broadcasted_iota(jnp.int32, (BLK, RADIAL_BINS), 1)
    r_sel = []
    for _ in range(TOP_R):
        v = jnp.min(dist_r, axis=1, keepdims=True)
        idx = jnp.min(jnp.where(dist_r == v, iota_r, RADIAL_BINS),
                      axis=1, keepdims=True)
        rc_v = jnp.max(jnp.where(iota_r == idx, rc, -jnp.inf),
                       axis=1, keepdims=True)
        r_sel.append((rc_v, idx))
        dist_r = jnp.where(iota_r == idx, jnp.inf, dist_r)

    a64 = a64_ref[...]
    dot = jax.lax.dot_general(x, a64, (((1,), (1,)), ((), ())),
                              precision=_HIGH,
                              preferred_element_type=jnp.float32)
    iota_w = jax.lax.broadcasted_iota(jnp.int32, (BLK, ANGULAR_BINS), 1)
    w_sel = []
    for _ in range(TOP_W):
        v = jnp.max(dot, axis=1, keepdims=True)
        idx = jnp.min(jnp.where(dot == v, iota_w, ANGULAR_BINS),
                      axis=1, keepdims=True)
        w_sel.append((v, idx))
        dot = jnp.where(iota_w == idx, -jnp.inf, dot)

    best_d = jnp.full((BLK, 1), jnp.inf, dtype=jnp.float32)
    best_ridx = jnp.zeros((BLK, 1), dtype=jnp.int32)
    best_widx = jnp.zeros((BLK, 1), dtype=jnp.int32)
    best_cosh = jnp.ones((BLK, 1), dtype=jnp.float32)
    best_sinh = jnp.zeros((BLK, 1), dtype=jnp.float32)
    for rc_v, ridx in r_sel:
        e = jnp.exp(rc_v)
        ch = 0.5 * (e + 1.0 / e)
        sh = 0.5 * (e - 1.0 / e)
        for dval, widx in w_sel:
            arg = jnp.maximum(u_t * ch - sh * dval, 1.0 + 1e-7)
            d = _acosh(arg)
            mask = d < best_d
            best_d = jnp.where(mask, d, best_d)
            best_ridx = jnp.where(mask, ridx, best_ridx)
            best_widx = jnp.where(mask, widx, best_widx)
            best_cosh = jnp.where(mask, ch, best_cosh)
            best_sinh = jnp.where(mask, sh, best_sinh)

    rx = _acosh(jnp.maximum(u_t, 1.0 + 1e-5))
    ry = _acosh(jnp.maximum(best_cosh, 1.0 + 1e-5))
    tad = best_d + jnp.abs(rx - ry)
    tad_part = jnp.sum(tad)

    oh_w = (iota_w == best_widx).astype(jnp.float32)
    zq = jax.lax.dot_general(oh_w, a64, (((1,), (0,)), ((), ())),
                             precision=_HIGH,
                             preferred_element_type=jnp.float32)
    col0 = (jax.lax.broadcasted_iota(jnp.int32, (BLK, E_DIM), 1) == 0)
    zq_ref[...] = zq * best_sinh + jnp.where(col0, best_cosh, 0.0)

    oh_r = (iota_r == best_ridx).astype(jnp.float32)
    hist_part = jax.lax.dot_general(oh_r, oh_w, (((0,), (0,)), ((), ())),
                                    precision=_HIGH,
                                    preferred_element_type=jnp.float32)

    @pl.when(i == 0)
    def _():
        hist_ref[...] = hist_part
        tad_ref[...] = tad_part.reshape(1, 1)

    @pl.when(i > 0)
    def _():
        hist_ref[...] += hist_part
        tad_ref[...] += tad_part.reshape(1, 1)

    @pl.when(i == nblk - 1)
    def _():
        e_mean = hist_ref[...] * (1.0 / n)
        emean_ref[...] = e_mean
        div = -jnp.sum(e_mean * jnp.log(e_mean + 1e-10))
        div_ref[...] = div.reshape(1, 1)
        perp_ref[...] = jnp.exp(div).reshape(1, 1)
        loss_ref[...] = (1.0 + BETA) * tad_ref[...] * (1.0 / n)


def kernel(u_hyp, r_centres, angular_weight):
    shape = u_hyp.shape
    flat = u_hyp.reshape(-1, shape[-1]).astype(jnp.float32)
    n = flat.shape[0]
    nblk = n // BLK
    a64 = jnp.concatenate(
        [jnp.zeros((ANGULAR_BINS, 1), jnp.float32),
         angular_weight.astype(jnp.float32)], axis=1)
    rc2d = r_centres.astype(jnp.float32).reshape(1, RADIAL_BINS)

    zq, hist, tad, loss, emean, div, perp = pl.pallas_call(
        functools.partial(_vq_kernel, nblk=nblk, n=n),
        grid=(nblk,),
        in_specs=[
            pl.BlockSpec((BLK, E_DIM), lambda i: (i, 0)),
            pl.BlockSpec((1, RADIAL_BINS), lambda i: (0, 0)),
            pl.BlockSpec((ANGULAR_BINS, E_DIM), lambda i: (0, 0)),
        ],
        out_specs=[
            pl.BlockSpec((BLK, E_DIM), lambda i: (i, 0)),
            pl.BlockSpec((RADIAL_BINS, ANGULAR_BINS), lambda i: (0, 0)),
            pl.BlockSpec((1, 1), lambda i: (0, 0)),
            pl.BlockSpec((1, 1), lambda i: (0, 0)),
            pl.BlockSpec((RADIAL_BINS, ANGULAR_BINS), lambda i: (0, 0)),
            pl.BlockSpec((1, 1), lambda i: (0, 0)),
            pl.BlockSpec((1, 1), lambda i: (0, 0)),
        ],
        out_shape=[
            jax.ShapeDtypeStruct((n, E_DIM), jnp.float32),
            jax.ShapeDtypeStruct((RADIAL_BINS, ANGULAR_BINS), jnp.float32),
            jax.ShapeDtypeStruct((1, 1), jnp.float32),
            jax.ShapeDtypeStruct((1, 1), jnp.float32),
            jax.ShapeDtypeStruct((RADIAL_BINS, ANGULAR_BINS), jnp.float32),
            jax.ShapeDtypeStruct((1, 1), jnp.float32),
            jax.ShapeDtypeStruct((1, 1), jnp.float32),
        ],
    )(flat, rc2d, a64)

    z_q = zq.reshape(shape)
    return (loss[0, 0], z_q, perp[0, 0], div[0, 0], emean.reshape(N_E))


# speedup vs baseline: 8.4572x; 1.7284x over previous
import functools

import jax
import jax.numpy as jnp
from jax.experimental import pallas as pl

N_E = 8192
E_DIM = 64
BETA = 0.25
RADIAL_BINS = 16
ANGULAR_BINS = N_E // RADIAL_BINS
MAX_RADIUS = 1.1
TOP_R = 3
TOP_W = 5
BLK = 1024

_HIGH = jax.lax.Precision.HIGHEST


def _acosh(x):
    return jnp.log(x + jnp.sqrt((x - 1.0) * (x + 1.0)))


def _vq_kernel(x_ref, rc_ref, a64_ref, zq_ref, hist_ref, tad_ref, loss_ref,
               emean_ref, div_ref, perp_ref, *, nblk, n):
    i = pl.program_id(0)
    x = x_ref[...]
    u_t = x[:, 0:1]
    r = _acosh(jnp.maximum(u_t, 1.01))

    rc = jnp.clip(rc_ref[...], 0.01, MAX_RADIUS)

    dist_r = jnp.abs(r - rc)
    iota_r = jax.lax.broadcasted_iota(jnp.int32, (BLK, RADIAL_BINS), 1)
    r_sel = []
    for _ in range(TOP_R):
        v = jnp.min(dist_r, axis=1, keepdims=True)
        idx = jnp.min(jnp.where(dist_r == v, iota_r, RADIAL_BINS),
                      axis=1, keepdims=True)
        rc_v = jnp.max(jnp.where(iota_r == idx, rc, -jnp.inf),
                       axis=1, keepdims=True)
        r_sel.append((rc_v, idx))
        dist_r = jnp.where(iota_r == idx, jnp.inf, dist_r)

    a64 = a64_ref[...]
    dot = jax.lax.dot_general(x, a64, (((1,), (1,)), ((), ())),
                              precision=_HIGH,
                              preferred_element_type=jnp.float32)
    iota_w = jax.lax.broadcasted_iota(jnp.int32, (BLK, ANGULAR_BINS), 1)
    dval = jnp.max(dot, axis=1, keepdims=True)
    best_widx = jnp.min(jnp.where(dot == dval, iota_w, ANGULAR_BINS),
                        axis=1, keepdims=True)

    best_d = jnp.full((BLK, 1), jnp.inf, dtype=jnp.float32)
    best_ridx = jnp.zeros((BLK, 1), dtype=jnp.int32)
    best_cosh = jnp.ones((BLK, 1), dtype=jnp.float32)
    best_sinh = jnp.zeros((BLK, 1), dtype=jnp.float32)
    for rc_v, ridx in r_sel:
        e = jnp.exp(rc_v)
        ch = 0.5 * (e + 1.0 / e)
        sh = 0.5 * (e - 1.0 / e)
        arg = jnp.maximum(u_t * ch - sh * dval, 1.0 + 1e-7)
        d = _acosh(arg)
        mask = d < best_d
        best_d = jnp.where(mask, d, best_d)
        best_ridx = jnp.where(mask, ridx, best_ridx)
        best_cosh = jnp.where(mask, ch, best_cosh)
        best_sinh = jnp.where(mask, sh, best_sinh)

    rx = _acosh(jnp.maximum(u_t, 1.0 + 1e-5))
    ry = _acosh(jnp.maximum(best_cosh, 1.0 + 1e-5))
    tad = best_d + jnp.abs(rx - ry)
    tad_part = jnp.sum(tad)

    oh_w = (iota_w == best_widx).astype(jnp.float32)
    zq = jax.lax.dot_general(oh_w, a64, (((1,), (0,)), ((), ())),
                             precision=_HIGH,
                             preferred_element_type=jnp.float32)
    col0 = (jax.lax.broadcasted_iota(jnp.int32, (BLK, E_DIM), 1) == 0)
    zq_ref[...] = zq * best_sinh + jnp.where(col0, best_cosh, 0.0)

    oh_r = (iota_r == best_ridx).astype(jnp.float32)
    hist_part = jax.lax.dot_general(oh_r, oh_w, (((0,), (0,)), ((), ())),
                                    precision=_HIGH,
                                    preferred_element_type=jnp.float32)

    @pl.when(i == 0)
    def _():
        hist_ref[...] = hist_part
        tad_ref[...] = tad_part.reshape(1, 1)

    @pl.when(i > 0)
    def _():
        hist_ref[...] += hist_part
        tad_ref[...] += tad_part.reshape(1, 1)

    @pl.when(i == nblk - 1)
    def _():
        e_mean = hist_ref[...] * (1.0 / n)
        emean_ref[...] = e_mean
        div = -jnp.sum(e_mean * jnp.log(e_mean + 1e-10))
        div_ref[...] = div.reshape(1, 1)
        perp_ref[...] = jnp.exp(div).reshape(1, 1)
        loss_ref[...] = (1.0 + BETA) * tad_ref[...] * (1.0 / n)


def kernel(u_hyp, r_centres, angular_weight):
    shape = u_hyp.shape
    flat = u_hyp.reshape(-1, shape[-1]).astype(jnp.float32)
    n = flat.shape[0]
    nblk = n // BLK
    a64 = jnp.concatenate(
        [jnp.zeros((ANGULAR_BINS, 1), jnp.float32),
         angular_weight.astype(jnp.float32)], axis=1)
    rc2d = r_centres.astype(jnp.float32).reshape(1, RADIAL_BINS)

    zq, hist, tad, loss, emean, div, perp = pl.pallas_call(
        functools.partial(_vq_kernel, nblk=nblk, n=n),
        grid=(nblk,),
        in_specs=[
            pl.BlockSpec((BLK, E_DIM), lambda i: (i, 0)),
            pl.BlockSpec((1, RADIAL_BINS), lambda i: (0, 0)),
            pl.BlockSpec((ANGULAR_BINS, E_DIM), lambda i: (0, 0)),
        ],
        out_specs=[
            pl.BlockSpec((BLK, E_DIM), lambda i: (i, 0)),
            pl.BlockSpec((RADIAL_BINS, ANGULAR_BINS), lambda i: (0, 0)),
            pl.BlockSpec((1, 1), lambda i: (0, 0)),
            pl.BlockSpec((1, 1), lambda i: (0, 0)),
            pl.BlockSpec((RADIAL_BINS, ANGULAR_BINS), lambda i: (0, 0)),
            pl.BlockSpec((1, 1), lambda i: (0, 0)),
            pl.BlockSpec((1, 1), lambda i: (0, 0)),
        ],
        out_shape=[
            jax.ShapeDtypeStruct((n, E_DIM), jnp.float32),
            jax.ShapeDtypeStruct((RADIAL_BINS, ANGULAR_BINS), jnp.float32),
            jax.ShapeDtypeStruct((1, 1), jnp.float32),
            jax.ShapeDtypeStruct((1, 1), jnp.float32),
            jax.ShapeDtypeStruct((RADIAL_BINS, ANGULAR_BINS), jnp.float32),
            jax.ShapeDtypeStruct((1, 1), jnp.float32),
            jax.ShapeDtypeStruct((1, 1), jnp.float32),
        ],
    )(flat, rc2d, a64)

    z_q = zq.reshape(shape)
    return (loss[0, 0], z_q, perp[0, 0], div[0, 0], emean.reshape(N_E))


# speedup vs baseline: 19.0783x; 2.2559x over previous
import functools

import jax
import jax.numpy as jnp
from jax.experimental import pallas as pl

N_E = 8192
E_DIM = 64
BETA = 0.25
RADIAL_BINS = 16
ANGULAR_BINS = N_E // RADIAL_BINS
MAX_RADIUS = 1.1
A_PAD = ANGULAR_BINS + 8
BLK = 1024

_HIGH = jax.lax.Precision.HIGHEST


def _acosh(x):
    return jnp.log(x + jnp.sqrt((x - 1.0) * (x + 1.0)))


def _vq_kernel(x_ref, rc_ref, a_ref, at_hi_ref, at_lo_ref,
               zqt_ref, hist_ref, tad_ref,
               loss_ref, emean_ref, div_ref, perp_ref, *, nblk, n):
    i = pl.program_id(0)
    x = x_ref[...]

    dotm = jax.lax.dot_general(a_ref[...], x, (((1,), (1,)), ((), ())),
                               precision=jax.lax.Precision.HIGHEST,
                               preferred_element_type=jnp.float32)
    u_t = dotm[ANGULAR_BINS:ANGULAR_BINS + 1, :]
    r = _acosh(jnp.maximum(u_t, 1.01))

    dot_v = dotm[:ANGULAR_BINS]
    iota_w = jax.lax.broadcasted_iota(jnp.int32, (ANGULAR_BINS, BLK), 0)
    dval = jnp.max(dot_v, axis=0, keepdims=True)
    best_widx = jnp.min(jnp.where(dot_v == dval, iota_w, ANGULAR_BINS),
                        axis=0, keepdims=True)

    rc = jnp.clip(rc_ref[...], 0.01, MAX_RADIUS)
    dist_r = jnp.abs(r - rc)
    iota_r = jax.lax.broadcasted_iota(jnp.int32, (RADIAL_BINS, BLK), 0)
    r_sel = []
    for _ in range(3):
        v = jnp.min(dist_r, axis=0, keepdims=True)
        idx = jnp.min(jnp.where(dist_r == v, iota_r, RADIAL_BINS),
                      axis=0, keepdims=True)
        rc_v = jnp.max(jnp.where(iota_r == idx, rc + 0.0 * dist_r, -jnp.inf),
                       axis=0, keepdims=True)
        r_sel.append((rc_v, idx))
        dist_r = jnp.where(iota_r == idx, jnp.inf, dist_r)

    best_d = jnp.full((1, BLK), jnp.inf, dtype=jnp.float32)
    best_ridx = jnp.zeros((1, BLK), dtype=jnp.int32)
    best_cosh = jnp.ones((1, BLK), dtype=jnp.float32)
    best_sinh = jnp.zeros((1, BLK), dtype=jnp.float32)
    for rc_v, ridx in r_sel:
        e = jnp.exp(rc_v)
        ch = 0.5 * (e + 1.0 / e)
        sh = 0.5 * (e - 1.0 / e)
        arg = jnp.maximum(u_t * ch - sh * dval, 1.0 + 1e-7)
        d = _acosh(arg)
        mask = d < best_d
        best_d = jnp.where(mask, d, best_d)
        best_ridx = jnp.where(mask, ridx, best_ridx)
        best_cosh = jnp.where(mask, ch, best_cosh)
        best_sinh = jnp.where(mask, sh, best_sinh)

    rx = _acosh(jnp.maximum(u_t, 1.0 + 1e-5))
    ry = _acosh(jnp.maximum(best_cosh, 1.0 + 1e-5))
    tad_part = jnp.sum(best_d + jnp.abs(rx - ry))

    oh_w = (iota_w == best_widx).astype(jnp.bfloat16)
    dims = (((1,), (0,)), ((), ()))
    zqt = jax.lax.dot_general(at_hi_ref[...], oh_w, dims,
                              preferred_element_type=jnp.float32)
    zqt += jax.lax.dot_general(at_lo_ref[...], oh_w, dims,
                               preferred_element_type=jnp.float32)
    e0 = (jax.lax.broadcasted_iota(jnp.int32, (E_DIM, BLK), 0) == 0)
    zqt_ref[...] = zqt * best_sinh + jnp.where(e0, best_cosh, 0.0)

    oh_r = (iota_r == best_ridx).astype(jnp.bfloat16)
    hist_part = jax.lax.dot_general(
        oh_r, oh_w, (((1,), (1,)), ((), ())),
        preferred_element_type=jnp.float32)

    @pl.when(i == 0)
    def _():
        hist_ref[...] = hist_part
        tad_ref[...] = tad_part.reshape(1, 1)

    @pl.when(i > 0)
    def _():
        hist_ref[...] += hist_part
        tad_ref[...] += tad_part.reshape(1, 1)

    @pl.when(i == nblk - 1)
    def _():
        e_mean = hist_ref[...] * (1.0 / n)
        emean_ref[...] = e_mean
        div = -jnp.sum(e_mean * jnp.log(e_mean + 1e-10))
        div_ref[...] = div.reshape(1, 1)
        perp_ref[...] = jnp.exp(div).reshape(1, 1)
        loss_ref[...] = (1.0 + BETA) * tad_ref[...] * (1.0 / n)


def kernel(u_hyp, r_centres, angular_weight):
    shape = u_hyp.shape
    flat = u_hyp.reshape(-1, shape[-1]).astype(jnp.float32)
    n = flat.shape[0]
    nblk = n // BLK
    aw = angular_weight.astype(jnp.float32)
    a = jnp.zeros((A_PAD, E_DIM), jnp.float32)
    a = a.at[:ANGULAR_BINS, 1:].set(aw)
    a = a.at[ANGULAR_BINS, 0].set(1.0)
    az = a[:ANGULAR_BINS]
    az_hi = az.astype(jnp.bfloat16)
    az_lo = (az - az_hi.astype(jnp.float32)).astype(jnp.bfloat16)
    at_hi, at_lo = az_hi.T, az_lo.T
    rc2d = r_centres.astype(jnp.float32).reshape(RADIAL_BINS, 1)

    zqt, hist, tad, loss, emean, div, perp = pl.pallas_call(
        functools.partial(_vq_kernel, nblk=nblk, n=n),
        grid=(nblk,),
        in_specs=[
            pl.BlockSpec((BLK, E_DIM), lambda i: (i, 0)),
            pl.BlockSpec((RADIAL_BINS, 1), lambda i: (0, 0)),
            pl.BlockSpec((A_PAD, E_DIM), lambda i: (0, 0)),
            pl.BlockSpec((E_DIM, ANGULAR_BINS), lambda i: (0, 0)),
            pl.BlockSpec((E_DIM, ANGULAR_BINS), lambda i: (0, 0)),
        ],
        out_specs=[
            pl.BlockSpec((E_DIM, BLK), lambda i: (0, i)),
            pl.BlockSpec((RADIAL_BINS, ANGULAR_BINS), lambda i: (0, 0)),
            pl.BlockSpec((1, 1), lambda i: (0, 0)),
            pl.BlockSpec((1, 1), lambda i: (0, 0)),
            pl.BlockSpec((RADIAL_BINS, ANGULAR_BINS), lambda i: (0, 0)),
            pl.BlockSpec((1, 1), lambda i: (0, 0)),
            pl.BlockSpec((1, 1), lambda i: (0, 0)),
        ],
        out_shape=[
            jax.ShapeDtypeStruct((E_DIM, n), jnp.float32),
            jax.ShapeDtypeStruct((RADIAL_BINS, ANGULAR_BINS), jnp.float32),
            jax.ShapeDtypeStruct((1, 1), jnp.float32),
            jax.ShapeDtypeStruct((1, 1), jnp.float32),
            jax.ShapeDtypeStruct((RADIAL_BINS, ANGULAR_BINS), jnp.float32),
            jax.ShapeDtypeStruct((1, 1), jnp.float32),
            jax.ShapeDtypeStruct((1, 1), jnp.float32),
        ],
    )(flat, rc2d, a, at_hi, at_lo)

    z_q = zqt.T.reshape(shape)
    return (loss[0, 0], z_q, perp[0, 0], div[0, 0], emean.reshape(N_E))
